# baseline (speedup 1.0000x reference)
import functools

import jax
import jax.numpy as jnp
from jax import lax
from jax.experimental import pallas as pl
from jax.experimental.pallas import tpu as pltpu
from jax.experimental.pallas import tpu_sc as plsc


def _make_lookup(B, L, D, interpret=False):
    NC, NS = 2, 16
    NW = NC * NS
    assert B % NW == 0 and D == 64 and L == 200
    seq_per_w = B // NW
    n_chunks = seq_per_w // 2
    T = 2 * L
    GB = 80
    NG = T // GB
    R2 = T // 2
    mesh = plsc.VectorSubcoreMesh(core_axis_name="c", subcore_axis_name="s",
                                  num_cores=NC, num_subcores=NS)

    @functools.partial(
        pl.kernel,
        out_type=jax.ShapeDtypeStruct((B * L // 2, 2 * D), jnp.float32),
        mesh=mesh,
        scratch_types=[
            pltpu.VMEM((T,), jnp.int32),
            pltpu.VMEM((NG, GB), jnp.int32),
            pltpu.VMEM((T + 16,), jnp.int32),
            pltpu.VMEM((T, 2 * D), jnp.float32),
            pltpu.VMEM((R2, 2 * D), jnp.float32),
            pltpu.VMEM((L // 2, 2 * D), jnp.float32),
            pltpu.SemaphoreType.DMA,
        ],
        interpret=interpret,
        name="wte_wpe_lookup",
    )
    def lookup(ids_hbm, wte2_hbm, wpe2_hbm, out_hbm,
               idx_v, rows_v, off_v, big_v, out_v, wpe_v, sem):
        wid = lax.axis_index("s") * NC + lax.axis_index("c")

        pltpu.sync_copy(wpe2_hbm, wpe_v)

        def chunk_body(c, carry):
            s0 = wid * seq_per_w + 2 * c
            base_tok = s0 * L
            pltpu.sync_copy(ids_hbm.at[pl.ds(base_tok, T)], idx_v)

            def prep_body(j, c2):
                g = j // (GB // 16)
                jj = j % (GB // 16)
                v = idx_v[pl.ds(j * 16, 16)]
                rows_v[g, pl.ds(jj * 16, 16)] = lax.shift_right_logical(v, 1)
                off_v[pl.ds(j * 16, 16)] = (v & 1) * (2 * D // 2)
                return c2

            for j in range(T // 16):
                prep_body(j, 0)

            copies = []
            for g in range(NG):
                copies.append(pltpu.async_copy(
                    wte2_hbm.at[rows_v.at[g]],
                    big_v.at[pl.ds(g * GB, GB)], sem))
            for cp in copies:
                cp.wait()

            def add_body(r, c2):
                a = 2 * r
                ov = off_v[pl.ds(a, 16)]
                off_a = ov[0]
                off_b = ov[1]
                for j in range(D // 16):
                    sl_lo = pl.ds(j * 16, 16)
                    sl_hi = pl.ds(D + j * 16, 16)
                    out_v[r, sl_lo] = (big_v[a, pl.ds(off_a + j * 16, 16)]
                                       + wpe_v[r % (L // 2), sl_lo])
                    out_v[r, sl_hi] = (big_v[a + 1, pl.ds(off_b + j * 16, 16)]
                                       + wpe_v[r % (L // 2), sl_hi])
                return c2

            lax.fori_loop(0, R2, add_body, 0, unroll=2)

            pltpu.sync_copy(out_v, out_hbm.at[pl.ds(s0 * (L // 2), R2)])
            return carry

        lax.fori_loop(0, n_chunks, chunk_body, 0)

    return lookup


def kernel(input_ids, wte_table, wpe_table):
    B, L = input_ids.shape
    V, D = wte_table.shape
    ids_flat = input_ids.reshape(B * L).astype(jnp.int32)
    wte2 = wte_table.reshape(V // 2, 2 * D)
    wpe2 = wpe_table[:L].reshape(L // 2, 2 * D)
    out2 = _make_lookup(B, L, D)(ids_flat, wte2, wpe2)
    return out2.reshape(B, L, D)
